# baseline (speedup 1.0000x reference)
import functools

import jax
import jax.numpy as jnp
from jax import lax
from jax.experimental import pallas as pl
from jax.experimental.pallas import tpu as pltpu
from jax.experimental.pallas import tpu_sc as plsc

N_VOCAB = 49408
D_MODEL = 768
SEQ_LEN = 77
BATCH = 1024
ROWS = BATCH * SEQ_LEN

_info = plsc.get_sparse_core_info()
_NC = _info.num_cores
_NS = _info.num_subcores
_NW = _NC * _NS
_RPW = ROWS // _NW
_CHUNK = 56
_NCHUNK = _RPW // _CHUNK

_mesh = plsc.VectorSubcoreMesh(core_axis_name="c", subcore_axis_name="s")


@functools.partial(
    pl.kernel,
    mesh=_mesh,
    out_type=jax.ShapeDtypeStruct((ROWS, D_MODEL), jnp.float32),
    scratch_types=[
        pltpu.VMEM((_CHUNK,), jnp.int32),
        pltpu.VMEM((_CHUNK, D_MODEL), jnp.float32),
        pltpu.VMEM((SEQ_LEN, D_MODEL), jnp.float32),
        pltpu.SemaphoreType.DMA,
    ],
)
def _clip_embed(tok_hbm, table_hbm, pos_hbm, out_hbm, idx_v, buf_v, pos_v, sem):
    wid = lax.axis_index("s") * _NC + lax.axis_index("c")
    wbase = wid * _RPW
    pltpu.sync_copy(pos_hbm, pos_v)

    def chunk_body(c, carry):
        base = wbase + c * _CHUNK
        pltpu.sync_copy(tok_hbm.at[pl.ds(base, _CHUNK)], idx_v)
        pltpu.async_copy(table_hbm.at[idx_v], buf_v, sem).wait()

        p0 = lax.rem(c * _CHUNK, SEQ_LEN)

        def row_body(r, c2):
            p = p0 + r
            p = jnp.where(p >= SEQ_LEN, p - SEQ_LEN, p)
            for dblk in range(D_MODEL // 16):
                sl = pl.ds(dblk * 16, 16)
                buf_v[r, sl] = buf_v[r, sl] + pos_v[p, sl]
            return c2

        lax.fori_loop(0, _CHUNK, row_body, 0)
        pltpu.sync_copy(buf_v, out_hbm.at[pl.ds(base, _CHUNK)])
        return carry

    lax.fori_loop(0, _NCHUNK, chunk_body, 0)


def kernel(tokens, tok_table, pos_table):
    out = _clip_embed(tokens.reshape(-1), tok_table, pos_table)
    return out.reshape(BATCH, SEQ_LEN, D_MODEL)


# speedup vs baseline: 1.2081x; 1.2081x over previous
import functools

import jax
import jax.numpy as jnp
from jax import lax
from jax.experimental import pallas as pl
from jax.experimental.pallas import tpu as pltpu
from jax.experimental.pallas import tpu_sc as plsc

N_VOCAB = 49408
D_MODEL = 768
SEQ_LEN = 77
BATCH = 1024
ROWS = BATCH * SEQ_LEN

_info = plsc.get_sparse_core_info()
_NC = _info.num_cores
_NS = _info.num_subcores
_NW = _NC * _NS
_RPW = ROWS // _NW
_CH = 16
_NCHUNK = _RPW // _CH

_mesh = plsc.VectorSubcoreMesh(core_axis_name="c", subcore_axis_name="s")


@functools.partial(
    pl.kernel,
    mesh=_mesh,
    out_type=jax.ShapeDtypeStruct((ROWS, D_MODEL), jnp.float32),
    scratch_types=[
        pltpu.VMEM((_RPW,), jnp.int32),
        pltpu.VMEM((_CH, D_MODEL), jnp.float32),
        pltpu.VMEM((_CH, D_MODEL), jnp.float32),
        pltpu.VMEM((_CH, D_MODEL), jnp.float32),
        pltpu.VMEM((_CH, D_MODEL), jnp.float32),
        pltpu.VMEM((SEQ_LEN, D_MODEL), jnp.float32),
        pltpu.SemaphoreType.DMA,
        pltpu.SemaphoreType.DMA,
        pltpu.SemaphoreType.DMA,
        pltpu.SemaphoreType.DMA,
    ],
)
def _clip_embed(tok_hbm, table_hbm, pos_hbm, out_hbm,
                idx_v, gb0, gb1, ob0, ob1, pos_v,
                gs0, gs1, ws0, ws1):
    wid = lax.axis_index("s") * _NC + lax.axis_index("c")
    wbase = wid * _RPW
    gbufs = (gb0, gb1)
    obufs = (ob0, ob1)
    gsems = (gs0, gs1)
    wsems = (ws0, ws1)

    pltpu.sync_copy(tok_hbm.at[pl.ds(wbase, _RPW)], idx_v)
    pltpu.sync_copy(pos_hbm, pos_v)

    def gather_start(c, par):
        off = pl.multiple_of(c * _CH, _CH)
        pltpu.make_async_copy(
            table_hbm.at[idx_v.at[pl.ds(off, _CH)]], gbufs[par], gsems[par]
        ).start()

    def gather_wait(par):
        pltpu.make_async_copy(
            table_hbm.at[idx_v.at[pl.ds(0, _CH)]], gbufs[par], gsems[par]
        ).wait()

    def write_start(c, par):
        off = pl.multiple_of(wbase + c * _CH, _CH)
        pltpu.make_async_copy(
            obufs[par], out_hbm.at[pl.ds(off, _CH)], wsems[par]
        ).start()

    def write_wait(par):
        pltpu.make_async_copy(
            obufs[par], out_hbm.at[pl.ds(0, _CH)], wsems[par]
        ).wait()

    def compute(c, par):
        p0 = lax.rem(c * _CH, SEQ_LEN)
        gb, ob = gbufs[par], obufs[par]

        def row_body(r, acc):
            p = p0 + r
            p = jnp.where(p >= SEQ_LEN, p - SEQ_LEN, p)
            for dblk in range(D_MODEL // 16):
                sl = pl.ds(dblk * 16, 16)
                ob[r, sl] = gb[r, sl] + pos_v[p, sl]
            return acc

        lax.fori_loop(0, _CH, row_body, 0)

    gather_start(0, 0)
    gather_start(1, 1)
    for c in (0, 1):
        gather_wait(c)
        compute(c, c)
        gather_start(c + 2, c)
        write_start(c, c)

    def pair_body(k, carry):
        for par in (0, 1):
            c = 2 * k + par
            gather_wait(par)
            write_wait(par)
            compute(c, par)
            gather_start(c + 2, par)
            write_start(c, par)
        return carry

    lax.fori_loop(1, _NCHUNK // 2 - 1, pair_body, 0)

    for c in (_NCHUNK - 2, _NCHUNK - 1):
        par = c % 2
        gather_wait(par)
        write_wait(par)
        compute(c, par)
        write_start(c, par)
    write_wait(0)
    write_wait(1)


def kernel(tokens, tok_table, pos_table):
    out = _clip_embed(tokens.reshape(-1), tok_table, pos_table)
    return out.reshape(BATCH, SEQ_LEN, D_MODEL)
